# baseline (speedup 1.0000x reference)
import functools

import jax
import jax.numpy as jnp
from jax import lax
from jax.experimental import pallas as pl
from jax.experimental.pallas import tpu as pltpu
from jax.experimental.pallas import tpu_sc as plsc

N = 10000
D = 128
E = 320000

NW = 32
CH = 64
CPW = 160
NBUF = 2
PIECE = 32
CPT0 = 160
CPT1 = 160
E0 = 16 * CPT0 * CH
EPW = CH * CPW
EPAD = EPW * NW
NP = 10112
RPT = NP // 16

_mesh = plsc.VectorSubcoreMesh(core_axis_name="c", subcore_axis_name="s")


def _zero_vmem_rows(ref, nrows, ncols):
    z = jnp.zeros((16,), jnp.float32)

    def body(i, _):
        for j in range(ncols // 16):
            ref[i, pl.ds(j * 16, 16)] = z
        return 0

    lax.fori_loop(0, nrows, body, 0)


def _zero_acc_slice(acc, zsrc, r0, znr):
    full, rem = RPT // znr, RPT % znr
    for t in range(full):
        pltpu.sync_copy(zsrc, acc.at[pl.ds(r0 + t * znr, znr)])
    if rem:
        pltpu.sync_copy(zsrc.at[pl.ds(0, rem)], acc.at[pl.ds(r0 + full * znr, rem)])


@functools.partial(
    pl.kernel,
    mesh=_mesh,
    out_type=jax.ShapeDtypeStruct((2, NP, D), jnp.float32),
    scratch_types=[
        pltpu.VMEM((CPW, CH), jnp.int32),
        pltpu.VMEM((CH, D), jnp.float32),
        pltpu.VMEM_SHARED((NP, D), jnp.float32),
    ],
)
def _deg(dst_hbm, out_hbm, di, vals, acc):
    c = lax.axis_index("c")
    s = lax.axis_index("s")
    wid = c * 16 + s

    _zero_vmem_rows(vals, CH, D)
    _zero_acc_slice(acc, vals, s * RPT, CH)
    plsc.subcore_barrier()

    e0 = jnp.where(lax.iota(jnp.int32, 16) == 0, 1.0, 0.0).astype(jnp.float32)

    def init(i, _):
        vals[i, pl.ds(0, 16)] = e0
        return 0

    lax.fori_loop(0, CH, init, 0)

    pltpu.sync_copy(dst_hbm.at[wid], di)

    def body(k, _):
        pltpu.sync_copy(vals, acc.at[di.at[k]], add=True)
        return 0

    lax.fori_loop(0, CPW, body, 0)
    plsc.subcore_barrier()

    r0 = s * RPT
    pltpu.sync_copy(acc.at[pl.ds(r0, RPT)], out_hbm.at[c, pl.ds(r0, RPT)])


@functools.partial(
    pl.kernel,
    mesh=_mesh,
    out_type=jax.ShapeDtypeStruct((2, NP, D), jnp.float32),
    scratch_types=[
        pltpu.VMEM((PIECE, CH), jnp.int32),
        pltpu.VMEM((PIECE, CH), jnp.int32),
    ]
    + [pltpu.VMEM((CH, D), jnp.float32) for _ in range(NBUF)]
    + [pltpu.SemaphoreType.DMA for _ in range(NBUF)]
    + [pltpu.VMEM_SHARED((NP, D), jnp.float32)],
)
def _agg(y_hbm, srcA, dstA, srcB, dstB, out_hbm, si, di, *rest):
    rows = rest[:NBUF]
    sems = rest[NBUF:2 * NBUF]
    acc = rest[2 * NBUF]
    c = lax.axis_index("c")
    s = lax.axis_index("s")

    _zero_vmem_rows(rows[0], CH, D)
    _zero_acc_slice(acc, rows[0], s * RPT, CH)
    plsc.subcore_barrier()

    def g_start(k, b):
        pltpu.make_async_copy(y_hbm.at[si.at[k]], rows[b], sems[b]).start()

    def g_wait(k, b):
        pltpu.make_async_copy(y_hbm.at[si.at[k]], rows[b], sems[b]).wait()

    def run(src_hbm, dst_hbm, cpt):
        for p0 in range(0, cpt, PIECE):
            pltpu.sync_copy(src_hbm.at[s, pl.ds(p0, PIECE)], si)
            pltpu.sync_copy(dst_hbm.at[s, pl.ds(p0, PIECE)], di)

            for b in range(NBUF):
                g_start(b, b)

            def body(g, _):
                for b in range(NBUF):
                    k = NBUF * g + b
                    g_wait(k, b)
                    pltpu.sync_copy(rows[b], acc.at[di.at[k]], add=True)
                    g_start(k + NBUF, b)
                return 0

            lax.fori_loop(0, (PIECE - NBUF) // NBUF, body, 0)
            for b in range(NBUF):
                k = PIECE - NBUF + b
                g_wait(k, b)
                pltpu.sync_copy(rows[b], acc.at[di.at[k]], add=True)

    @pl.when(c == 0)
    def _work0():
        run(srcA, dstA, CPT0)

    @pl.when(c == 1)
    def _work1():
        run(srcB, dstB, CPT1)

    plsc.subcore_barrier()
    r0 = s * RPT
    pltpu.sync_copy(acc.at[pl.ds(r0, RPT)], out_hbm.at[c, pl.ds(r0, RPT)])


BR = 1000


def _dinv_col(degp_ref):
    deg = degp_ref[0, :, 0] + degp_ref[1, :, 0] + 1.0
    return lax.rsqrt(deg)[:, None]


def _tc1_body(x_ref, w_ref, degp_ref, y_ref):
    y = jnp.dot(x_ref[...], w_ref[...], preferred_element_type=jnp.float32)
    y_ref[...] = y * _dinv_col(degp_ref)


def _tc_mid_body(aggp_ref, y_ref, degp_ref, w_ref, b_ref, out_ref):
    dinv = _dinv_col(degp_ref)
    tot = (aggp_ref[0] + aggp_ref[1] + y_ref[...]) * dinv + b_ref[...][None, :]
    h = jnp.maximum(tot, 0.0)
    out_ref[...] = jnp.dot(h, w_ref[...], preferred_element_type=jnp.float32) * dinv


def _tc4_body(aggp_ref, y_ref, degp_ref, b_ref, out_ref):
    dinv = _dinv_col(degp_ref)
    o = (aggp_ref[0] + aggp_ref[1] + y_ref[...]) * dinv + b_ref[...][None, :]
    m = jnp.max(o, axis=1, keepdims=True)
    lse = jnp.log(jnp.sum(jnp.exp(o - m), axis=1, keepdims=True)) + m
    out_ref[...] = o - lse


_spec_rows = pl.BlockSpec((BR, D), lambda i: (i, 0))
_spec_w = pl.BlockSpec((D, D), lambda i: (0, 0))
_spec_b = pl.BlockSpec((D,), lambda i: (0,))
_spec_deg = pl.BlockSpec((2, BR, D), lambda i: (0, i, 0))
_spec_agg = pl.BlockSpec((2, BR, D), lambda i: (0, i, 0))


def _tc1(x, W, degp):
    return pl.pallas_call(
        _tc1_body,
        grid=(N // BR,),
        in_specs=[_spec_rows, _spec_w, _spec_deg],
        out_specs=_spec_rows,
        out_shape=jax.ShapeDtypeStruct((NP, D), jnp.float32),
    )(x, W, degp)


def _tc_mid(aggp, y, degp, W, b):
    return pl.pallas_call(
        _tc_mid_body,
        grid=(N // BR,),
        in_specs=[_spec_agg, _spec_rows, _spec_deg, _spec_w, _spec_b],
        out_specs=_spec_rows,
        out_shape=jax.ShapeDtypeStruct((NP, D), jnp.float32),
    )(aggp, y, degp, W, b)


def _tc4(aggp, y, degp, b):
    return pl.pallas_call(
        _tc4_body,
        grid=(N // BR,),
        in_specs=[_spec_agg, _spec_rows, _spec_deg, _spec_b],
        out_specs=_spec_rows,
        out_shape=jax.ShapeDtypeStruct((N, D), jnp.float32),
    )(aggp, y, degp, b)


def kernel(x, edge_index, W1, b1, Wh, bh, W2, b2):
    pad = N + (jnp.arange(EPAD - E, dtype=jnp.int32) % (NP - N))
    src_all = jnp.concatenate([edge_index[0], pad])
    dst_all = jnp.concatenate([edge_index[1], pad])
    dst3 = dst_all.reshape(NW, CPW, CH)
    srcA = src_all[:E0].reshape(16, CPT0, CH)
    dstA = dst_all[:E0].reshape(16, CPT0, CH)
    srcB = src_all[E0:].reshape(16, CPT1, CH)
    dstB = dst_all[E0:].reshape(16, CPT1, CH)

    degp = _deg(dst3)
    y1 = _tc1(x, W1, degp)
    a1 = _agg(y1, srcA, dstA, srcB, dstB)
    y2 = _tc_mid(a1, y1, degp, Wh, b1)
    a2 = _agg(y2, srcA, dstA, srcB, dstB)
    y3 = _tc_mid(a2, y2, degp, W2, bh)
    a3 = _agg(y3, srcA, dstA, srcB, dstB)
    return _tc4(a3, y3, degp, b2)


# speedup vs baseline: 1.0786x; 1.0786x over previous
import functools

import jax
import jax.numpy as jnp
from jax import lax
from jax.experimental import pallas as pl
from jax.experimental.pallas import tpu as pltpu
from jax.experimental.pallas import tpu_sc as plsc

N = 10000
D = 128
E = 320000

NW = 32
CH = 32
CPW = 320
NBUF = 4
PIECE = 64
CPT0 = 320
CPT1 = 320
E0 = 16 * CPT0 * CH
EPW = CH * CPW
EPAD = EPW * NW
NP = 10112
RPT = NP // 16

_mesh = plsc.VectorSubcoreMesh(core_axis_name="c", subcore_axis_name="s")


def _zero_vmem_rows(ref, nrows, ncols):
    z = jnp.zeros((16,), jnp.float32)

    def body(i, _):
        for j in range(ncols // 16):
            ref[i, pl.ds(j * 16, 16)] = z
        return 0

    lax.fori_loop(0, nrows, body, 0)


def _zero_acc_slice(acc, zsrc, r0, znr):
    full, rem = RPT // znr, RPT % znr
    for t in range(full):
        pltpu.sync_copy(zsrc, acc.at[pl.ds(r0 + t * znr, znr)])
    if rem:
        pltpu.sync_copy(zsrc.at[pl.ds(0, rem)], acc.at[pl.ds(r0 + full * znr, rem)])


@functools.partial(
    pl.kernel,
    mesh=_mesh,
    out_type=jax.ShapeDtypeStruct((2, NP, D), jnp.float32),
    scratch_types=[
        pltpu.VMEM((CPW, CH), jnp.int32),
        pltpu.VMEM((CH, D), jnp.float32),
        pltpu.VMEM_SHARED((NP, D), jnp.float32),
    ],
)
def _deg(dst_hbm, out_hbm, di, vals, acc):
    c = lax.axis_index("c")
    s = lax.axis_index("s")
    wid = c * 16 + s

    _zero_vmem_rows(vals, CH, D)
    _zero_acc_slice(acc, vals, s * RPT, CH)
    plsc.subcore_barrier()

    e0 = jnp.where(lax.iota(jnp.int32, 16) == 0, 1.0, 0.0).astype(jnp.float32)

    def init(i, _):
        vals[i, pl.ds(0, 16)] = e0
        return 0

    lax.fori_loop(0, CH, init, 0)

    pltpu.sync_copy(dst_hbm.at[wid], di)

    def body(k, _):
        pltpu.sync_copy(vals, acc.at[di.at[k]], add=True)
        return 0

    lax.fori_loop(0, CPW, body, 0)
    plsc.subcore_barrier()

    r0 = s * RPT
    pltpu.sync_copy(acc.at[pl.ds(r0, RPT)], out_hbm.at[c, pl.ds(r0, RPT)])


@functools.partial(
    pl.kernel,
    mesh=_mesh,
    out_type=jax.ShapeDtypeStruct((2, NP, D), jnp.float32),
    scratch_types=[
        pltpu.VMEM((PIECE, CH), jnp.int32),
        pltpu.VMEM((PIECE, CH), jnp.int32),
    ]
    + [pltpu.VMEM((CH, D), jnp.float32) for _ in range(NBUF)]
    + [pltpu.SemaphoreType.DMA for _ in range(NBUF)]
    + [pltpu.VMEM_SHARED((NP, D), jnp.float32)],
)
def _agg(y_hbm, srcA, dstA, srcB, dstB, out_hbm, si, di, *rest):
    rows = rest[:NBUF]
    sems = rest[NBUF:2 * NBUF]
    acc = rest[2 * NBUF]
    c = lax.axis_index("c")
    s = lax.axis_index("s")

    _zero_vmem_rows(rows[0], CH, D)
    _zero_acc_slice(acc, rows[0], s * RPT, CH)
    plsc.subcore_barrier()

    def g_start(k, b):
        pltpu.make_async_copy(y_hbm.at[si.at[k]], rows[b], sems[b]).start()

    def g_wait(k, b):
        pltpu.make_async_copy(y_hbm.at[si.at[k]], rows[b], sems[b]).wait()

    def run(src_hbm, dst_hbm, cpt):
        for p0 in range(0, cpt, PIECE):
            pltpu.sync_copy(src_hbm.at[s, pl.ds(p0, PIECE)], si)
            pltpu.sync_copy(dst_hbm.at[s, pl.ds(p0, PIECE)], di)

            for b in range(NBUF):
                g_start(b, b)

            def body(g, _):
                for b in range(NBUF):
                    k = NBUF * g + b
                    g_wait(k, b)
                    pltpu.sync_copy(rows[b], acc.at[di.at[k]], add=True)
                    g_start(k + NBUF, b)
                return 0

            lax.fori_loop(0, (PIECE - NBUF) // NBUF, body, 0)
            for b in range(NBUF):
                k = PIECE - NBUF + b
                g_wait(k, b)
                pltpu.sync_copy(rows[b], acc.at[di.at[k]], add=True)

    @pl.when(c == 0)
    def _work0():
        run(srcA, dstA, CPT0)

    @pl.when(c == 1)
    def _work1():
        run(srcB, dstB, CPT1)

    plsc.subcore_barrier()
    r0 = s * RPT
    pltpu.sync_copy(acc.at[pl.ds(r0, RPT)], out_hbm.at[c, pl.ds(r0, RPT)])


BR = 1000


def _tc1_body(x_ref, w_ref, degp_ref, y_ref, dinv_ref):
    deg = degp_ref[0, :, 0] + degp_ref[1, :, 0] + 1.0
    dinv = lax.rsqrt(deg)
    dinv_ref[...] = dinv[None, None, :]
    y = jnp.dot(x_ref[...], w_ref[...], preferred_element_type=jnp.float32)
    y_ref[...] = y * dinv[:, None]


def _tc_mid_body(aggp_ref, y_ref, dinv_ref, w_ref, b_ref, out_ref):
    dinv = dinv_ref[0, 0][:, None]
    tot = (aggp_ref[0] + aggp_ref[1] + y_ref[...]) * dinv + b_ref[...][None, :]
    h = jnp.maximum(tot, 0.0)
    out_ref[...] = jnp.dot(h, w_ref[...], preferred_element_type=jnp.float32) * dinv


def _tc4_body(aggp_ref, y_ref, dinv_ref, b_ref, out_ref):
    dinv = dinv_ref[0, 0][:, None]
    o = (aggp_ref[0] + aggp_ref[1] + y_ref[...]) * dinv + b_ref[...][None, :]
    m = jnp.max(o, axis=1, keepdims=True)
    lse = jnp.log(jnp.sum(jnp.exp(o - m), axis=1, keepdims=True)) + m
    out_ref[...] = o - lse


_spec_rows = pl.BlockSpec((BR, D), lambda i: (i, 0))
_spec_w = pl.BlockSpec((D, D), lambda i: (0, 0))
_spec_b = pl.BlockSpec((D,), lambda i: (0,))
_spec_deg = pl.BlockSpec((2, BR, D), lambda i: (0, i, 0))
_spec_agg = pl.BlockSpec((2, BR, D), lambda i: (0, i, 0))
_spec_dinv = pl.BlockSpec((1, 1, BR), lambda i: (i, 0, 0))


def _tc1(x, W, degp):
    return pl.pallas_call(
        _tc1_body,
        grid=(N // BR,),
        in_specs=[_spec_rows, _spec_w, _spec_deg],
        out_specs=[_spec_rows, _spec_dinv],
        out_shape=[
            jax.ShapeDtypeStruct((NP, D), jnp.float32),
            jax.ShapeDtypeStruct((N // BR, 1, BR), jnp.float32),
        ],
    )(x, W, degp)


def _tc_mid(aggp, y, dinv, W, b):
    return pl.pallas_call(
        _tc_mid_body,
        grid=(N // BR,),
        in_specs=[_spec_agg, _spec_rows, _spec_dinv, _spec_w, _spec_b],
        out_specs=_spec_rows,
        out_shape=jax.ShapeDtypeStruct((NP, D), jnp.float32),
    )(aggp, y, dinv, W, b)


def _tc4(aggp, y, dinv, b):
    return pl.pallas_call(
        _tc4_body,
        grid=(N // BR,),
        in_specs=[_spec_agg, _spec_rows, _spec_dinv, _spec_b],
        out_specs=_spec_rows,
        out_shape=jax.ShapeDtypeStruct((N, D), jnp.float32),
    )(aggp, y, dinv, b)


def kernel(x, edge_index, W1, b1, Wh, bh, W2, b2):
    pad = N + (jnp.arange(EPAD - E, dtype=jnp.int32) % (NP - N))
    src_all = jnp.concatenate([edge_index[0], pad])
    dst_all = jnp.concatenate([edge_index[1], pad])
    dst3 = dst_all.reshape(NW, CPW, CH)
    srcA = src_all[:E0].reshape(16, CPT0, CH)
    dstA = dst_all[:E0].reshape(16, CPT0, CH)
    srcB = src_all[E0:].reshape(16, CPT1, CH)
    dstB = dst_all[E0:].reshape(16, CPT1, CH)

    degp = _deg(dst3)
    y1, dinv = _tc1(x, W1, degp)
    a1 = _agg(y1, srcA, dstA, srcB, dstB)
    y2 = _tc_mid(a1, y1, dinv, Wh, b1)
    a2 = _agg(y2, srcA, dstA, srcB, dstB)
    y3 = _tc_mid(a2, y2, dinv, W2, bh)
    a3 = _agg(y3, srcA, dstA, srcB, dstB)
    return _tc4(a3, y3, dinv, b2)


# speedup vs baseline: 1.1925x; 1.1057x over previous
import functools

import jax
import jax.numpy as jnp
from jax import lax
from jax.experimental import pallas as pl
from jax.experimental.pallas import tpu as pltpu
from jax.experimental.pallas import tpu_sc as plsc

N = 10000
D = 128
E = 320000

NW = 32
CH = 32
CPW = 320
NBUF = 6
PIECE = 64
CPT0 = 320
CPT1 = 320
E0 = 16 * CPT0 * CH
EPW = CH * CPW
EPAD = EPW * NW
NP = 10112
RPT = NP // 16

_mesh = plsc.VectorSubcoreMesh(core_axis_name="c", subcore_axis_name="s")


def _zero_vmem_rows(ref, nrows, ncols):
    z = jnp.zeros((16,), jnp.float32)

    def body(i, _):
        for j in range(ncols // 16):
            ref[i, pl.ds(j * 16, 16)] = z
        return 0

    lax.fori_loop(0, nrows, body, 0)


def _zero_acc_slice(acc, zsrc, r0, znr):
    full, rem = RPT // znr, RPT % znr
    for t in range(full):
        pltpu.sync_copy(zsrc, acc.at[pl.ds(r0 + t * znr, znr)])
    if rem:
        pltpu.sync_copy(zsrc.at[pl.ds(0, rem)], acc.at[pl.ds(r0 + full * znr, rem)])


@functools.partial(
    pl.kernel,
    mesh=_mesh,
    out_type=jax.ShapeDtypeStruct((2, NP, D), jnp.float32),
    scratch_types=[
        pltpu.VMEM((CPW, CH), jnp.int32),
        pltpu.VMEM((CH, D), jnp.float32),
        pltpu.VMEM_SHARED((NP, D), jnp.float32),
    ],
)
def _deg(dst_hbm, out_hbm, di, vals, acc):
    c = lax.axis_index("c")
    s = lax.axis_index("s")
    wid = c * 16 + s

    _zero_vmem_rows(vals, CH, D)
    _zero_acc_slice(acc, vals, s * RPT, CH)
    plsc.subcore_barrier()

    e0 = jnp.where(lax.iota(jnp.int32, 16) == 0, 1.0, 0.0).astype(jnp.float32)

    def init(i, _):
        vals[i, pl.ds(0, 16)] = e0
        return 0

    lax.fori_loop(0, CH, init, 0)

    pltpu.sync_copy(dst_hbm.at[wid], di)

    def body(k, _):
        pltpu.sync_copy(vals, acc.at[di.at[k]], add=True)
        return 0

    lax.fori_loop(0, CPW, body, 0)
    plsc.subcore_barrier()

    r0 = s * RPT
    pltpu.sync_copy(acc.at[pl.ds(r0, RPT)], out_hbm.at[c, pl.ds(r0, RPT)])


@functools.partial(
    pl.kernel,
    mesh=_mesh,
    out_type=jax.ShapeDtypeStruct((2, NP, D), jnp.float32),
    scratch_types=[
        pltpu.VMEM((PIECE, CH), jnp.int32),
        pltpu.VMEM((PIECE, CH), jnp.int32),
    ]
    + [pltpu.VMEM((CH, D), jnp.float32) for _ in range(NBUF)]
    + [pltpu.SemaphoreType.DMA for _ in range(NBUF)]
    + [pltpu.VMEM_SHARED((NP, D), jnp.float32)],
)
def _agg(y_hbm, srcA, dstA, srcB, dstB, out_hbm, si, di, *rest):
    rows = rest[:NBUF]
    sems = rest[NBUF:2 * NBUF]
    acc = rest[2 * NBUF]
    c = lax.axis_index("c")
    s = lax.axis_index("s")

    _zero_vmem_rows(rows[0], CH, D)
    _zero_acc_slice(acc, rows[0], s * RPT, CH)
    plsc.subcore_barrier()

    def g_start(k, b):
        pltpu.make_async_copy(y_hbm.at[si.at[k]], rows[b], sems[b]).start()

    def g_wait(k, b):
        pltpu.make_async_copy(y_hbm.at[si.at[k]], rows[b], sems[b]).wait()

    def run(src_hbm, dst_hbm, cpt):
        for p0 in range(0, cpt, PIECE):
            pltpu.sync_copy(src_hbm.at[s, pl.ds(p0, PIECE)], si)
            pltpu.sync_copy(dst_hbm.at[s, pl.ds(p0, PIECE)], di)

            for b in range(NBUF):
                g_start(b, b)

            def body(g, _):
                for b in range(NBUF):
                    k = NBUF * g + b
                    g_wait(k, b)
                    pltpu.sync_copy(rows[b], acc.at[di.at[k]], add=True)
                    g_start(k + NBUF, b)
                return 0

            lax.fori_loop(0, (PIECE - NBUF) // NBUF, body, 0)
            for b in range(NBUF):
                k = PIECE - NBUF + b
                g_wait(k, b)
                pltpu.sync_copy(rows[b], acc.at[di.at[k]], add=True)

    @pl.when(c == 0)
    def _work0():
        run(srcA, dstA, CPT0)

    @pl.when(c == 1)
    def _work1():
        run(srcB, dstB, CPT1)

    plsc.subcore_barrier()
    r0 = s * RPT
    pltpu.sync_copy(acc.at[pl.ds(r0, RPT)], out_hbm.at[c, pl.ds(r0, RPT)])


BR = 1000


def _tc1_body(x_ref, w_ref, degp_ref, y_ref, dinv_ref):
    deg = degp_ref[0, :, 0] + degp_ref[1, :, 0] + 1.0
    dinv = lax.rsqrt(deg)
    dinv_ref[...] = dinv[None, None, :]
    y = jnp.dot(x_ref[...], w_ref[...], preferred_element_type=jnp.float32)
    y_ref[...] = y * dinv[:, None]


def _tc_mid_body(aggp_ref, y_ref, dinv_ref, w_ref, b_ref, out_ref):
    dinv = dinv_ref[0, 0][:, None]
    tot = (aggp_ref[0] + aggp_ref[1] + y_ref[...]) * dinv + b_ref[...][None, :]
    h = jnp.maximum(tot, 0.0)
    out_ref[...] = jnp.dot(h, w_ref[...], preferred_element_type=jnp.float32) * dinv


def _tc4_body(aggp_ref, y_ref, dinv_ref, b_ref, out_ref):
    dinv = dinv_ref[0, 0][:, None]
    o = (aggp_ref[0] + aggp_ref[1] + y_ref[...]) * dinv + b_ref[...][None, :]
    m = jnp.max(o, axis=1, keepdims=True)
    lse = jnp.log(jnp.sum(jnp.exp(o - m), axis=1, keepdims=True)) + m
    out_ref[...] = o - lse


_spec_rows = pl.BlockSpec((BR, D), lambda i: (i, 0))
_spec_w = pl.BlockSpec((D, D), lambda i: (0, 0))
_spec_b = pl.BlockSpec((D,), lambda i: (0,))
_spec_deg = pl.BlockSpec((2, BR, D), lambda i: (0, i, 0))
_spec_agg = pl.BlockSpec((2, BR, D), lambda i: (0, i, 0))
_spec_dinv = pl.BlockSpec((1, 1, BR), lambda i: (i, 0, 0))


def _tc1(x, W, degp):
    return pl.pallas_call(
        _tc1_body,
        grid=(N // BR,),
        in_specs=[_spec_rows, _spec_w, _spec_deg],
        out_specs=[_spec_rows, _spec_dinv],
        out_shape=[
            jax.ShapeDtypeStruct((NP, D), jnp.float32),
            jax.ShapeDtypeStruct((N // BR, 1, BR), jnp.float32),
        ],
    )(x, W, degp)


def _tc_mid(aggp, y, dinv, W, b):
    return pl.pallas_call(
        _tc_mid_body,
        grid=(N // BR,),
        in_specs=[_spec_agg, _spec_rows, _spec_dinv, _spec_w, _spec_b],
        out_specs=_spec_rows,
        out_shape=jax.ShapeDtypeStruct((NP, D), jnp.float32),
    )(aggp, y, dinv, W, b)


def _tc4(aggp, y, dinv, b):
    return pl.pallas_call(
        _tc4_body,
        grid=(N // BR,),
        in_specs=[_spec_agg, _spec_rows, _spec_dinv, _spec_b],
        out_specs=_spec_rows,
        out_shape=jax.ShapeDtypeStruct((N, D), jnp.float32),
    )(aggp, y, dinv, b)


def kernel(x, edge_index, W1, b1, Wh, bh, W2, b2):
    pad = N + (jnp.arange(EPAD - E, dtype=jnp.int32) % (NP - N))
    src_all = jnp.concatenate([edge_index[0], pad])
    dst_all = jnp.concatenate([edge_index[1], pad])
    dst3 = dst_all.reshape(NW, CPW, CH)
    srcA = src_all[:E0].reshape(16, CPT0, CH)
    dstA = dst_all[:E0].reshape(16, CPT0, CH)
    srcB = src_all[E0:].reshape(16, CPT1, CH)
    dstB = dst_all[E0:].reshape(16, CPT1, CH)

    degp = _deg(dst3)
    y1, dinv = _tc1(x, W1, degp)
    a1 = _agg(y1, srcA, dstA, srcB, dstB)
    y2 = _tc_mid(a1, y1, dinv, Wh, b1)
    a2 = _agg(y2, srcA, dstA, srcB, dstB)
    y3 = _tc_mid(a2, y2, dinv, W2, bh)
    a3 = _agg(y3, srcA, dstA, srcB, dstB)
    return _tc4(a3, y3, dinv, b2)
